# Initial kernel scaffold; baseline (speedup 1.0000x reference)
import jax
import jax.numpy as jnp
from jax.experimental import pallas as pl


def kernel(x, edge_index, z_in, z_out):
    raise NotImplementedError("write your pallas kernel here")


# speedup vs baseline: 4.2977x; 4.2977x over previous
import functools

import jax
import jax.numpy as jnp
from jax import lax
from jax.experimental import pallas as pl
from jax.experimental.pallas import tpu as pltpu
from jax.experimental.pallas import tpu_sc as plsc

_MAX_DEG = 64
_D = 256
_N = 10000
_E2 = 2 * 160000

_BN = 512
_NB_HALF = 10240
_NBINS = 2 * _NB_HALF
_NW = 32
_CHUNK = _E2 // _NW

@functools.cache
def _make_sc_hist():
    mesh = plsc.VectorSubcoreMesh(core_axis_name="c", subcore_axis_name="s",
                                  num_cores=2, num_subcores=16)
    return pl.kernel(
        _sc_hist_body,
        out_type=jax.ShapeDtypeStruct((_NW, _NBINS), jnp.int32),
        mesh=mesh,
        scratch_types=[
            pltpu.VMEM((_CHUNK,), jnp.int32),
            pltpu.VMEM((_NBINS,), jnp.int32),
        ],
        compiler_params=pltpu.CompilerParams(needs_layout_passes=False),
    )


def _sc_hist_body(ef_hbm, out_hbm, ev, hist):
    nc = 2
    wid = lax.axis_index("s") * nc + lax.axis_index("c")
    base = wid * _CHUNK
    pltpu.sync_copy(ef_hbm.at[pl.ds(base, _CHUNK)], ev)

    def zero_body(i, carry):
        hist[pl.ds(i * 16, 16)] = jnp.zeros((16,), jnp.int32)
        return carry

    lax.fori_loop(0, _NBINS // 16, zero_body, 0)

    offs = (lax.iota(jnp.int32, 16) % 2) * _NB_HALF
    ones = jnp.ones((16,), jnp.int32)

    def body(i, carry):
        v = ev[pl.ds(i * 16, 16)]
        plsc.addupdate_scatter(hist, [v + offs], ones)
        return carry

    lax.fori_loop(0, _CHUNK // 16, body, 0)
    pltpu.sync_copy(hist, out_hbm.at[wid])


def _tc_body(pt_out_ref, pt_in_ref, zin_ref, zout_ref, x_ref, o_ref):
    i = pl.program_id(0)
    dout = jnp.minimum(jnp.sum(pt_out_ref[...], axis=1, keepdims=True),
                       _MAX_DEG - 1)
    din = jnp.minimum(jnp.sum(pt_in_ref[...], axis=1, keepdims=True),
                      _MAX_DEG - 1)
    it = lax.broadcasted_iota(jnp.int32, (_BN, _MAX_DEG), 1)
    oh_out = (it == dout).astype(jnp.float32)
    oh_in = (it == din).astype(jnp.float32)
    add = (
        jnp.dot(oh_in, zin_ref[...], preferred_element_type=jnp.float32,
                precision=lax.Precision.HIGHEST)
        + jnp.dot(oh_out, zout_ref[...], preferred_element_type=jnp.float32,
                  precision=lax.Precision.HIGHEST)
    )
    nid = i * _BN + lax.broadcasted_iota(jnp.int32, (_BN, 1), 0)
    add = jnp.where(nid > 0, add, 0.0)
    o_ref[...] = x_ref[...] + add[None, :, :]


def _tc_apply(pt_out, pt_in, z_in, z_out, x):
    nblk = _NB_HALF // _BN
    return pl.pallas_call(
        _tc_body,
        grid=(nblk,),
        in_specs=[
            pl.BlockSpec((_BN, _NW), lambda i: (i, 0)),
            pl.BlockSpec((_BN, _NW), lambda i: (i, 0)),
            pl.BlockSpec((_MAX_DEG, _D), lambda i: (0, 0)),
            pl.BlockSpec((_MAX_DEG, _D), lambda i: (0, 0)),
            pl.BlockSpec((x.shape[0], _BN, _D), lambda i: (0, i, 0)),
        ],
        out_specs=pl.BlockSpec((x.shape[0], _BN, _D), lambda i: (0, i, 0)),
        out_shape=jax.ShapeDtypeStruct(x.shape, x.dtype),
    )(pt_out, pt_in, z_in, z_out, x)


def kernel(x, edge_index, z_in, z_out):
    ef = edge_index.reshape(-1)
    partials = _make_sc_hist()(ef)
    pt = partials.T
    pt_out = pt[:_NB_HALF]
    pt_in = pt[_NB_HALF:]
    return _tc_apply(pt_out, pt_in, z_in, z_out, x)


# speedup vs baseline: 4.3296x; 1.0074x over previous
import functools

import jax
import jax.numpy as jnp
from jax import lax
from jax.experimental import pallas as pl
from jax.experimental.pallas import tpu as pltpu
from jax.experimental.pallas import tpu_sc as plsc

_MAX_DEG = 64
_D = 256
_N = 10000
_E2 = 2 * 160000

_BN = 1024
_NB_HALF = 10240
_NBINS = 2 * _NB_HALF
_NW = 32
_CHUNK = _E2 // _NW

@functools.cache
def _make_sc_hist():
    mesh = plsc.VectorSubcoreMesh(core_axis_name="c", subcore_axis_name="s",
                                  num_cores=2, num_subcores=16)
    return pl.kernel(
        _sc_hist_body,
        out_type=jax.ShapeDtypeStruct((_NW, _NBINS), jnp.int32),
        mesh=mesh,
        scratch_types=[
            pltpu.VMEM((_CHUNK,), jnp.int32),
            pltpu.VMEM((_NBINS,), jnp.int32),
        ],
        compiler_params=pltpu.CompilerParams(needs_layout_passes=False),
    )


def _sc_hist_body(ef_hbm, out_hbm, ev, hist):
    nc = 2
    wid = lax.axis_index("s") * nc + lax.axis_index("c")
    base = wid * _CHUNK
    pltpu.sync_copy(ef_hbm.at[pl.ds(base, _CHUNK)], ev)

    def zero_body(i, carry):
        hist[pl.ds(i * 16, 16)] = jnp.zeros((16,), jnp.int32)
        return carry

    lax.fori_loop(0, _NBINS // 16, zero_body, 0)

    offs = (lax.iota(jnp.int32, 16) % 2) * _NB_HALF
    ones = jnp.ones((16,), jnp.int32)

    def body(i, carry):
        v = ev[pl.ds(i * 16, 16)]
        plsc.addupdate_scatter(hist, [v + offs], ones)
        return carry

    lax.fori_loop(0, _CHUNK // 16, body, 0)
    pltpu.sync_copy(hist, out_hbm.at[wid])


def _tc_body(pt_out_ref, pt_in_ref, zin_ref, zout_ref, x_ref, o_ref):
    i = pl.program_id(0)
    dout = jnp.minimum(jnp.sum(pt_out_ref[...], axis=1, keepdims=True),
                       _MAX_DEG - 1)
    din = jnp.minimum(jnp.sum(pt_in_ref[...], axis=1, keepdims=True),
                      _MAX_DEG - 1)
    it = lax.broadcasted_iota(jnp.int32, (_BN, _MAX_DEG), 1)
    oh_out = (it == dout).astype(jnp.float32)
    oh_in = (it == din).astype(jnp.float32)
    add = (
        jnp.dot(oh_in, zin_ref[...], preferred_element_type=jnp.float32,
                precision=lax.Precision.HIGHEST)
        + jnp.dot(oh_out, zout_ref[...], preferred_element_type=jnp.float32,
                  precision=lax.Precision.HIGHEST)
    )
    nid = i * _BN + lax.broadcasted_iota(jnp.int32, (_BN, 1), 0)
    add = jnp.where(nid > 0, add, 0.0)
    o_ref[...] = x_ref[...] + add[None, :, :]


def _tc_apply(pt_out, pt_in, z_in, z_out, x):
    nblk = _NB_HALF // _BN
    return pl.pallas_call(
        _tc_body,
        grid=(nblk,),
        in_specs=[
            pl.BlockSpec((_BN, _NW), lambda i: (i, 0)),
            pl.BlockSpec((_BN, _NW), lambda i: (i, 0)),
            pl.BlockSpec((_MAX_DEG, _D), lambda i: (0, 0)),
            pl.BlockSpec((_MAX_DEG, _D), lambda i: (0, 0)),
            pl.BlockSpec((x.shape[0], _BN, _D), lambda i: (0, i, 0)),
        ],
        out_specs=pl.BlockSpec((x.shape[0], _BN, _D), lambda i: (0, i, 0)),
        out_shape=jax.ShapeDtypeStruct(x.shape, x.dtype),
    )(pt_out, pt_in, z_in, z_out, x)


def kernel(x, edge_index, z_in, z_out):
    ef = edge_index.reshape(-1)
    partials = _make_sc_hist()(ef)
    pt = partials.T
    pt_out = pt[:_NB_HALF]
    pt_in = pt[_NB_HALF:]
    return _tc_apply(pt_out, pt_in, z_in, z_out, x)


# speedup vs baseline: 4.7593x; 1.0993x over previous
import functools

import jax
import jax.numpy as jnp
from jax import lax
from jax.experimental import pallas as pl
from jax.experimental.pallas import tpu as pltpu
from jax.experimental.pallas import tpu_sc as plsc

_MAX_DEG = 64
_D = 256
_N = 10000
_E2 = 2 * 160000

_BN = 1024
_NB_HALF = 10240
_NBINS = 2 * _NB_HALF
_NW = 32
_CHUNK = _E2 // _NW

@functools.cache
def _make_sc_hist():
    mesh = plsc.VectorSubcoreMesh(core_axis_name="c", subcore_axis_name="s",
                                  num_cores=2, num_subcores=16)
    return pl.kernel(
        _sc_hist_body,
        out_type=jax.ShapeDtypeStruct((_NW, _NBINS), jnp.int32),
        mesh=mesh,
        scratch_types=[
            pltpu.VMEM((_CHUNK,), jnp.int32),
            pltpu.VMEM((_NBINS,), jnp.int32),
        ],
        compiler_params=pltpu.CompilerParams(needs_layout_passes=False),
    )


def _sc_hist_body(ef_hbm, out_hbm, ev, hist):
    nc = 2
    wid = lax.axis_index("s") * nc + lax.axis_index("c")
    base = wid * _CHUNK
    pltpu.sync_copy(ef_hbm.at[pl.ds(base, _CHUNK)], ev)

    def zero_body(i, carry):
        hist[pl.ds(i * 16, 16)] = jnp.zeros((16,), jnp.int32)
        return carry

    lax.fori_loop(0, _NBINS // 16, zero_body, 0)

    offs = (lax.iota(jnp.int32, 16) % 2) * _NB_HALF
    ones = jnp.ones((16,), jnp.int32)

    def body(i, carry):
        v = ev[pl.ds(i * 16, 16)]
        plsc.addupdate_scatter(hist, [v + offs], ones)
        return carry

    lax.fori_loop(0, _CHUNK // 16, body, 0)
    pltpu.sync_copy(hist, out_hbm.at[wid])


def _tc_body(pt_out_ref, pt_in_ref, zin_ref, zout_ref, x_ref, o_ref):
    i = pl.program_id(0)
    dout = jnp.minimum(jnp.sum(pt_out_ref[...], axis=0, keepdims=True),
                       _MAX_DEG - 1)
    din = jnp.minimum(jnp.sum(pt_in_ref[...], axis=0, keepdims=True),
                      _MAX_DEG - 1)
    it = lax.broadcasted_iota(jnp.int32, (_MAX_DEG, _BN), 0)
    oh_out_t = (it == dout).astype(jnp.float32)
    oh_in_t = (it == din).astype(jnp.float32)
    dn = (((0,), (0,)), ((), ()))
    add = (
        lax.dot_general(oh_in_t, zin_ref[...], dn,
                        preferred_element_type=jnp.float32,
                        precision=lax.Precision.HIGHEST)
        + lax.dot_general(oh_out_t, zout_ref[...], dn,
                          preferred_element_type=jnp.float32,
                          precision=lax.Precision.HIGHEST)
    )
    nid = i * _BN + lax.broadcasted_iota(jnp.int32, (_BN, 1), 0)
    add = jnp.where(nid > 0, add, 0.0)
    o_ref[...] = x_ref[...] + add[None, :, :]


def _tc_apply(partials, z_in, z_out, x):
    nblk = _NB_HALF // _BN
    return pl.pallas_call(
        _tc_body,
        grid=(nblk,),
        in_specs=[
            pl.BlockSpec((_NW, _BN), lambda i: (0, i)),
            pl.BlockSpec((_NW, _BN), lambda i: (0, i + _NB_HALF // _BN)),
            pl.BlockSpec((_MAX_DEG, _D), lambda i: (0, 0)),
            pl.BlockSpec((_MAX_DEG, _D), lambda i: (0, 0)),
            pl.BlockSpec((x.shape[0], _BN, _D), lambda i: (0, i, 0)),
        ],
        out_specs=pl.BlockSpec((x.shape[0], _BN, _D), lambda i: (0, i, 0)),
        out_shape=jax.ShapeDtypeStruct(x.shape, x.dtype),
    )(partials, partials, z_in, z_out, x)


def kernel(x, edge_index, z_in, z_out):
    ef = edge_index.reshape(-1)
    partials = _make_sc_hist()(ef)
    return _tc_apply(partials, z_in, z_out, x)


# speedup vs baseline: 9.2016x; 1.9334x over previous
import functools

import jax
import jax.numpy as jnp
from jax import lax
from jax.experimental import pallas as pl
from jax.experimental.pallas import tpu as pltpu
from jax.experimental.pallas import tpu_sc as plsc

_MAX_DEG = 64
_D = 256
_N = 10000
_E = 160000

_BN = 1024
_NB_HALF = 10240
_HROWS = _NB_HALF // 128
_NW = 32
_CHUNK = _E // (_NW // 2)


@functools.cache
def _make_sc_hist():
    mesh = plsc.VectorSubcoreMesh(core_axis_name="c", subcore_axis_name="s",
                                  num_cores=2, num_subcores=16)
    return pl.kernel(
        _sc_hist_body,
        out_type=jax.ShapeDtypeStruct((_NW, 2 * _HROWS, 128), jnp.int32),
        mesh=mesh,
        scratch_types=[
            pltpu.VMEM((_CHUNK,), jnp.int32),
            pltpu.VMEM((2 * _HROWS, 128), jnp.int32),
        ],
        compiler_params=pltpu.CompilerParams(needs_layout_passes=False),
    )


def _sc_hist_body(src_hbm, dst_hbm, out_hbm, ev, hist):
    nc = 2
    wid = lax.axis_index("s") * nc + lax.axis_index("c")
    half = wid // 16
    base = (wid % 16) * _CHUNK

    @pl.when(half == 0)
    def _():
        pltpu.sync_copy(src_hbm.at[pl.ds(base, _CHUNK)], ev)

    @pl.when(half == 1)
    def _():
        pltpu.sync_copy(dst_hbm.at[pl.ds(base, _CHUNK)], ev)

    def zero_body(i, carry):
        hist[i >> 3, pl.ds((i & 7) * 16, 16)] = jnp.zeros((16,), jnp.int32)
        return carry

    lax.fori_loop(0, 2 * _HROWS * 8, zero_body, 0)

    row_off = half * _HROWS
    ones = jnp.ones((16,), jnp.int32)

    def body(i, carry):
        v = ev[pl.ds(i * 16, 16)]
        plsc.addupdate_scatter(hist, [(v >> 7) + row_off, v & 127], ones)
        return carry

    lax.fori_loop(0, _CHUNK // 16, body, 0)
    pltpu.sync_copy(hist, out_hbm.at[wid])


def _tc_body(pt_out_ref, pt_in_ref, zin_ref, zout_ref, x_ref, o_ref):
    i = pl.program_id(0)
    dout8 = jnp.minimum(jnp.sum(pt_out_ref[...], axis=0), _MAX_DEG - 1)
    din8 = jnp.minimum(jnp.sum(pt_in_ref[...], axis=0), _MAX_DEG - 1)
    it = lax.broadcasted_iota(jnp.int32, (_MAX_DEG, 128), 0)
    dn = (((0,), (0,)), ((), ()))
    adds = []
    for r in range(_BN // 128):
        oh_out_t = (it == dout8[r:r + 1, :]).astype(jnp.float32)
        oh_in_t = (it == din8[r:r + 1, :]).astype(jnp.float32)
        adds.append(
            lax.dot_general(oh_in_t, zin_ref[...], dn,
                            preferred_element_type=jnp.float32,
                            precision=lax.Precision.HIGHEST)
            + lax.dot_general(oh_out_t, zout_ref[...], dn,
                              preferred_element_type=jnp.float32,
                              precision=lax.Precision.HIGHEST)
        )
    add = jnp.concatenate(adds, axis=0)
    nid = i * _BN + lax.broadcasted_iota(jnp.int32, (_BN, 1), 0)
    add = jnp.where(nid > 0, add, 0.0)
    o_ref[...] = x_ref[...] + add[None, :, :]


def _tc_apply(partials, z_in, z_out, x):
    nblk = _NB_HALF // _BN
    rpb = _BN // 128
    return pl.pallas_call(
        _tc_body,
        grid=(nblk,),
        in_specs=[
            pl.BlockSpec((_NW, rpb, 128), lambda i: (0, i, 0)),
            pl.BlockSpec((_NW, rpb, 128), lambda i: (0, i + _HROWS // rpb, 0)),
            pl.BlockSpec((_MAX_DEG, _D), lambda i: (0, 0)),
            pl.BlockSpec((_MAX_DEG, _D), lambda i: (0, 0)),
            pl.BlockSpec((x.shape[0], _BN, _D), lambda i: (0, i, 0)),
        ],
        out_specs=pl.BlockSpec((x.shape[0], _BN, _D), lambda i: (0, i, 0)),
        out_shape=jax.ShapeDtypeStruct(x.shape, x.dtype),
    )(partials, partials, z_in, z_out, x)


def kernel(x, edge_index, z_in, z_out):
    src = edge_index[:, 0]
    dst = edge_index[:, 1]
    partials = _make_sc_hist()(src, dst)
    return _tc_apply(partials, z_in, z_out, x)


# speedup vs baseline: 9.6693x; 1.0508x over previous
import functools

import jax
import jax.numpy as jnp
from jax import lax
from jax.experimental import pallas as pl
from jax.experimental.pallas import tpu as pltpu
from jax.experimental.pallas import tpu_sc as plsc

_MAX_DEG = 64
_D = 256
_N = 10000
_E = 160000

_BN = 1024
_NB_HALF = 10240
_HROWS = _NB_HALF // 128
_NW = 32
_CHUNK = _E // (_NW // 2)


@functools.cache
def _make_sc_hist():
    mesh = plsc.VectorSubcoreMesh(core_axis_name="c", subcore_axis_name="s",
                                  num_cores=2, num_subcores=16)
    return pl.kernel(
        _sc_hist_body,
        out_type=jax.ShapeDtypeStruct((_NW, 2 * _HROWS, 128), jnp.int32),
        mesh=mesh,
        scratch_types=[
            pltpu.VMEM((_CHUNK,), jnp.int32),
            pltpu.VMEM((2 * _HROWS, 128), jnp.int32),
        ],
        compiler_params=pltpu.CompilerParams(needs_layout_passes=False),
    )


def _sc_hist_body(src_hbm, dst_hbm, out_hbm, ev, hist):
    nc = 2
    wid = lax.axis_index("s") * nc + lax.axis_index("c")
    half = wid // 16
    base = (wid % 16) * _CHUNK

    @pl.when(half == 0)
    def _():
        pltpu.sync_copy(src_hbm.at[pl.ds(base, _CHUNK)], ev)

    @pl.when(half == 1)
    def _():
        pltpu.sync_copy(dst_hbm.at[pl.ds(base, _CHUNK)], ev)

    zeros = jnp.zeros((16,), jnp.int32)

    def zero_body(i, carry):
        for u in range(8):
            hist[i, pl.ds(u * 16, 16)] = zeros
        return carry

    lax.fori_loop(0, 2 * _HROWS, zero_body, 0)

    row_off = half * _HROWS
    ones = jnp.ones((16,), jnp.int32)
    _U = 5

    def body(i, carry):
        for u in range(_U):
            v = ev[pl.ds((i * _U + u) * 16, 16)]
            plsc.addupdate_scatter(hist, [(v >> 7) + row_off, v & 127], ones)
        return carry

    lax.fori_loop(0, _CHUNK // 16 // _U, body, 0)
    pltpu.sync_copy(hist, out_hbm.at[wid])


def _tc_body(pt_out_ref, pt_in_ref, zin_ref, zout_ref, x_ref, o_ref):
    i = pl.program_id(0)
    dout8 = jnp.minimum(jnp.sum(pt_out_ref[...], axis=0), _MAX_DEG - 1)
    din8 = jnp.minimum(jnp.sum(pt_in_ref[...], axis=0), _MAX_DEG - 1)
    it = lax.broadcasted_iota(jnp.int32, (_MAX_DEG, 128), 0)
    dn = (((0,), (0,)), ((), ()))
    adds = []
    for r in range(_BN // 128):
        oh_out_t = (it == dout8[r:r + 1, :]).astype(jnp.float32)
        oh_in_t = (it == din8[r:r + 1, :]).astype(jnp.float32)
        adds.append(
            lax.dot_general(oh_in_t, zin_ref[...], dn,
                            preferred_element_type=jnp.float32,
                            precision=lax.Precision.HIGHEST)
            + lax.dot_general(oh_out_t, zout_ref[...], dn,
                              preferred_element_type=jnp.float32,
                              precision=lax.Precision.HIGHEST)
        )
    add = jnp.concatenate(adds, axis=0)
    nid = i * _BN + lax.broadcasted_iota(jnp.int32, (_BN, 1), 0)
    add = jnp.where(nid > 0, add, 0.0)
    o_ref[...] = x_ref[...] + add[None, :, :]


def _tc_apply(partials, z_in, z_out, x):
    nblk = _NB_HALF // _BN
    rpb = _BN // 128
    return pl.pallas_call(
        _tc_body,
        grid=(nblk,),
        in_specs=[
            pl.BlockSpec((_NW, rpb, 128), lambda i: (0, i, 0)),
            pl.BlockSpec((_NW, rpb, 128), lambda i: (0, i + _HROWS // rpb, 0)),
            pl.BlockSpec((_MAX_DEG, _D), lambda i: (0, 0)),
            pl.BlockSpec((_MAX_DEG, _D), lambda i: (0, 0)),
            pl.BlockSpec((x.shape[0], _BN, _D), lambda i: (0, i, 0)),
        ],
        out_specs=pl.BlockSpec((x.shape[0], _BN, _D), lambda i: (0, i, 0)),
        out_shape=jax.ShapeDtypeStruct(x.shape, x.dtype),
    )(partials, partials, z_in, z_out, x)


def kernel(x, edge_index, z_in, z_out):
    src = edge_index[:, 0]
    dst = edge_index[:, 1]
    partials = _make_sc_hist()(src, dst)
    return _tc_apply(partials, z_in, z_out, x)


# speedup vs baseline: 9.7666x; 1.0101x over previous
import functools

import jax
import jax.numpy as jnp
from jax import lax
from jax.experimental import pallas as pl
from jax.experimental.pallas import tpu as pltpu
from jax.experimental.pallas import tpu_sc as plsc

_MAX_DEG = 64
_D = 256
_N = 10000
_E = 160000

_BN = 1024
_NB_HALF = 10240
_HROWS = _NB_HALF // 128
_NW = 32
_CHUNK = _E // (_NW // 2)


@functools.cache
def _make_sc_hist():
    mesh = plsc.VectorSubcoreMesh(core_axis_name="c", subcore_axis_name="s",
                                  num_cores=2, num_subcores=16)
    return pl.kernel(
        _sc_hist_body,
        out_type=jax.ShapeDtypeStruct((_NW, 2 * _HROWS, 128), jnp.int32),
        mesh=mesh,
        scratch_types=[
            pltpu.VMEM((_CHUNK,), jnp.int32),
            pltpu.VMEM((2 * _HROWS, 128), jnp.int32),
            pltpu.SemaphoreType.DMA,
        ],
        compiler_params=pltpu.CompilerParams(needs_layout_passes=False),
    )


def _sc_hist_body(src_hbm, dst_hbm, out_hbm, ev, hist, sem):
    nc = 2
    wid = lax.axis_index("s") * nc + lax.axis_index("c")
    half = wid // 16
    base = (wid % 16) * _CHUNK

    @pl.when(half == 0)
    def _():
        pltpu.async_copy(src_hbm.at[pl.ds(base, _CHUNK)], ev, sem)

    @pl.when(half == 1)
    def _():
        pltpu.async_copy(dst_hbm.at[pl.ds(base, _CHUNK)], ev, sem)

    zeros = jnp.zeros((16,), jnp.int32)

    def zero_body(i, carry):
        for u in range(8):
            hist[i, pl.ds(u * 16, 16)] = zeros
        return carry

    lax.fori_loop(0, 2 * _HROWS, zero_body, 0)
    pltpu.make_async_copy(src_hbm.at[pl.ds(base, _CHUNK)], ev, sem).wait()

    row_off = half * _HROWS
    ones = jnp.ones((16,), jnp.int32)
    _U = 25

    def body(i, carry):
        for u in range(_U):
            v = ev[pl.ds((i * _U + u) * 16, 16)]
            plsc.addupdate_scatter(hist, [(v >> 7) + row_off, v & 127], ones)
        return carry

    lax.fori_loop(0, _CHUNK // 16 // _U, body, 0)
    pltpu.sync_copy(hist, out_hbm.at[wid])


def _tc_body(pt_out_ref, pt_in_ref, zin_ref, zout_ref, x_ref, o_ref):
    i = pl.program_id(0)
    dout8 = jnp.minimum(jnp.sum(pt_out_ref[...], axis=0), _MAX_DEG - 1)
    din8 = jnp.minimum(jnp.sum(pt_in_ref[...], axis=0), _MAX_DEG - 1)
    it = lax.broadcasted_iota(jnp.int32, (_MAX_DEG, 128), 0)
    dn = (((0,), (0,)), ((), ()))
    adds = []
    for r in range(_BN // 128):
        oh_out_t = (it == dout8[r:r + 1, :]).astype(jnp.float32)
        oh_in_t = (it == din8[r:r + 1, :]).astype(jnp.float32)
        adds.append(
            lax.dot_general(oh_in_t, zin_ref[...], dn,
                            preferred_element_type=jnp.float32,
                            precision=lax.Precision.HIGHEST)
            + lax.dot_general(oh_out_t, zout_ref[...], dn,
                              preferred_element_type=jnp.float32,
                              precision=lax.Precision.HIGHEST)
        )
    add = jnp.concatenate(adds, axis=0)
    nid = i * _BN + lax.broadcasted_iota(jnp.int32, (_BN, 1), 0)
    add = jnp.where(nid > 0, add, 0.0)
    o_ref[...] = x_ref[...] + add[None, :, :]


def _tc_apply(partials, z_in, z_out, x):
    nblk = _NB_HALF // _BN
    rpb = _BN // 128
    return pl.pallas_call(
        _tc_body,
        grid=(nblk,),
        in_specs=[
            pl.BlockSpec((_NW, rpb, 128), lambda i: (0, i, 0)),
            pl.BlockSpec((_NW, rpb, 128), lambda i: (0, i + _HROWS // rpb, 0)),
            pl.BlockSpec((_MAX_DEG, _D), lambda i: (0, 0)),
            pl.BlockSpec((_MAX_DEG, _D), lambda i: (0, 0)),
            pl.BlockSpec((x.shape[0], _BN, _D), lambda i: (0, i, 0)),
        ],
        out_specs=pl.BlockSpec((x.shape[0], _BN, _D), lambda i: (0, i, 0)),
        out_shape=jax.ShapeDtypeStruct(x.shape, x.dtype),
    )(partials, partials, z_in, z_out, x)


def kernel(x, edge_index, z_in, z_out):
    src = edge_index[:, 0]
    dst = edge_index[:, 1]
    partials = _make_sc_hist()(src, dst)
    return _tc_apply(partials, z_in, z_out, x)
